# baseline (speedup 1.0000x reference)
import functools

import jax
import jax.numpy as jnp
from jax import lax
from jax.experimental import pallas as pl
from jax.experimental.pallas import tpu as pltpu
from jax.experimental.pallas import tpu_sc as plsc

VOCAB = 100000
D = 64
B = 1024

NV = 5120


@functools.cache
def _make_gather():
    info = plsc.get_sparse_core_info()
    nw = info.num_cores * info.num_subcores
    rows_per_w = D // nw
    nvec = B // info.num_lanes
    mesh = plsc.VectorSubcoreMesh(core_axis_name="c", subcore_axis_name="s")

    @functools.partial(
        pl.kernel,
        mesh=mesh,
        compiler_params=pltpu.CompilerParams(needs_layout_passes=False),
        out_type=jax.ShapeDtypeStruct((D, B), jnp.float32),
        scratch_types=[
            pltpu.VMEM((B,), jnp.int32),
            pltpu.VMEM((VOCAB,), jnp.float32),
            pltpu.VMEM((B,), jnp.float32),
        ],
    )
    def gather_k(table_hbm, idx_hbm, out_hbm, idx_v, row_v, out_v):
        wid = lax.axis_index("s") * info.num_cores + lax.axis_index("c")
        pltpu.sync_copy(idx_hbm, idx_v)
        for r in range(rows_per_w):
            row = wid * rows_per_w + r
            pltpu.sync_copy(table_hbm.at[row], row_v)

            def body(k, _):
                i16 = idx_v[pl.ds(k * info.num_lanes, info.num_lanes)]
                vals = plsc.load_gather(row_v, [i16])
                out_v[pl.ds(k * info.num_lanes, info.num_lanes)] = vals
                return _

            lax.fori_loop(0, nvec, body, None)
            pltpu.sync_copy(out_v, out_hbm.at[row])

    return gather_k


def _mm_body(et_ref, w_ref, b_ref, out_ref):
    bias = lax.dot_general(
        b_ref[...].reshape(1, b_ref.shape[0]), jnp.ones((1, B), jnp.float32),
        dimension_numbers=(((0,), (0,)), ((), ())),
        preferred_element_type=jnp.float32,
    )
    out_ref[...] = lax.dot_general(
        w_ref[...], et_ref[...],
        dimension_numbers=(((0,), (0,)), ((), ())),
        preferred_element_type=jnp.float32,
    ) + bias


def _matmul(et, w_t, b):
    return pl.pallas_call(
        _mm_body,
        grid=(pl.cdiv(VOCAB, NV),),
        in_specs=[
            pl.BlockSpec((D, B), lambda j: (0, 0)),
            pl.BlockSpec((D, NV), lambda j: (0, j)),
            pl.BlockSpec((NV,), lambda j: (j,)),
        ],
        out_specs=pl.BlockSpec((NV, B), lambda j: (j, 0)),
        out_shape=jax.ShapeDtypeStruct((VOCAB, B), jnp.float32),
    )(et, w_t, b)


def kernel(x, emb_table, W, b):
    table_t = emb_table.T
    w_t = W.T
    et = _make_gather()(table_t, x.astype(jnp.int32))
    logits_t = _matmul(et, w_t, b)
    return (logits_t.T, et.T)


# speedup vs baseline: 1.0011x; 1.0011x over previous
import functools

import jax
import jax.numpy as jnp
from jax import lax
from jax.experimental import pallas as pl
from jax.experimental.pallas import tpu as pltpu
from jax.experimental.pallas import tpu_sc as plsc

VOCAB = 100000
D = 64
B = 1024

NV = 4096


@functools.cache
def _make_gather():
    info = plsc.get_sparse_core_info()
    nw = info.num_cores * info.num_subcores
    rows_per_w = D // nw
    nvec = B // info.num_lanes
    mesh = plsc.VectorSubcoreMesh(core_axis_name="c", subcore_axis_name="s")

    @functools.partial(
        pl.kernel,
        mesh=mesh,
        compiler_params=pltpu.CompilerParams(needs_layout_passes=False),
        out_type=jax.ShapeDtypeStruct((D, B), jnp.float32),
        scratch_types=[
            pltpu.VMEM((B,), jnp.int32),
            pltpu.VMEM((VOCAB,), jnp.float32),
            pltpu.VMEM((B,), jnp.float32),
        ],
    )
    def gather_k(table_hbm, idx_hbm, out_hbm, idx_v, row_v, out_v):
        wid = lax.axis_index("s") * info.num_cores + lax.axis_index("c")
        pltpu.sync_copy(idx_hbm, idx_v)
        for r in range(rows_per_w):
            row = wid * rows_per_w + r
            pltpu.sync_copy(table_hbm.at[row], row_v)

            def body(k, _):
                i16 = idx_v[pl.ds(k * info.num_lanes, info.num_lanes)]
                vals = plsc.load_gather(row_v, [i16])
                out_v[pl.ds(k * info.num_lanes, info.num_lanes)] = vals
                return _

            lax.fori_loop(0, nvec, body, None)
            pltpu.sync_copy(out_v, out_hbm.at[row])

    return gather_k


def _mm_body(et_ref, w_ref, b_ref, out_ref):
    bias = lax.dot_general(
        b_ref[...].reshape(1, b_ref.shape[0]), jnp.ones((1, B), jnp.float32),
        dimension_numbers=(((0,), (0,)), ((), ())),
        preferred_element_type=jnp.float32,
    )
    out_ref[...] = lax.dot_general(
        w_ref[...], et_ref[...],
        dimension_numbers=(((0,), (0,)), ((), ())),
        preferred_element_type=jnp.float32,
    ) + bias


def _matmul(et, w_t, b):
    return pl.pallas_call(
        _mm_body,
        grid=(pl.cdiv(VOCAB, NV),),
        in_specs=[
            pl.BlockSpec((D, B), lambda j: (0, 0)),
            pl.BlockSpec((D, NV), lambda j: (0, j)),
            pl.BlockSpec((NV,), lambda j: (j,)),
        ],
        out_specs=pl.BlockSpec((NV, B), lambda j: (j, 0)),
        out_shape=jax.ShapeDtypeStruct((VOCAB, B), jnp.float32),
    )(et, w_t, b)


def kernel(x, emb_table, W, b):
    table_t = emb_table.T
    w_t = W.T
    et = _make_gather()(table_t, x.astype(jnp.int32))
    logits_t = _matmul(et, w_t, b)
    return (logits_t.T, et.T)


# speedup vs baseline: 1.0026x; 1.0015x over previous
import functools

import jax
import jax.numpy as jnp
from jax import lax
from jax.experimental import pallas as pl
from jax.experimental.pallas import tpu as pltpu
from jax.experimental.pallas import tpu_sc as plsc

VOCAB = 100000
D = 64
B = 1024

NV = 4096


@functools.cache
def _make_gather():
    info = plsc.get_sparse_core_info()
    nw = info.num_cores * info.num_subcores
    rows_per_w = D // nw
    nvec = B // info.num_lanes
    mesh = plsc.VectorSubcoreMesh(core_axis_name="c", subcore_axis_name="s")

    @functools.partial(
        pl.kernel,
        mesh=mesh,
        compiler_params=pltpu.CompilerParams(needs_layout_passes=False),
        out_type=jax.ShapeDtypeStruct((D, B), jnp.float32),
        scratch_types=[
            pltpu.VMEM((B,), jnp.int32),
            pltpu.VMEM((VOCAB,), jnp.float32),
            pltpu.VMEM((B,), jnp.float32),
        ],
    )
    def gather_k(table_hbm, idx_hbm, out_hbm, idx_v, row_v, out_v):
        wid = lax.axis_index("s") * info.num_cores + lax.axis_index("c")
        pltpu.sync_copy(idx_hbm, idx_v)
        for r in range(rows_per_w):
            row = wid * rows_per_w + r
            pltpu.sync_copy(table_hbm.at[row], row_v)

            def body(k, _):
                i16 = idx_v[pl.ds(k * info.num_lanes, info.num_lanes)]
                vals = plsc.load_gather(row_v, [i16])
                out_v[pl.ds(k * info.num_lanes, info.num_lanes)] = vals
                return _

            lax.fori_loop(0, nvec, body, None)
            pltpu.sync_copy(out_v, out_hbm.at[row])

    return gather_k


def _mm_body(et_ref, w_ref, b_ref, out_ref):
    bias = lax.dot_general(
        b_ref[...].reshape(1, b_ref.shape[0]), jnp.ones((1, B), jnp.float32),
        dimension_numbers=(((0,), (0,)), ((), ())),
        preferred_element_type=jnp.float32,
    )
    out_ref[...] = lax.dot_general(
        w_ref[...], et_ref[...],
        dimension_numbers=(((0,), (0,)), ((), ())),
        preferred_element_type=jnp.float32,
    ) + bias


def _matmul(et, w_t, b):
    return pl.pallas_call(
        _mm_body,
        grid=(pl.cdiv(VOCAB, NV),),
        compiler_params=pltpu.CompilerParams(fuse_transposed_lhs_in_matmul=True),
        in_specs=[
            pl.BlockSpec((D, B), lambda j: (0, 0)),
            pl.BlockSpec((D, NV), lambda j: (0, j)),
            pl.BlockSpec((NV,), lambda j: (j,)),
        ],
        out_specs=pl.BlockSpec((NV, B), lambda j: (j, 0)),
        out_shape=jax.ShapeDtypeStruct((VOCAB, B), jnp.float32),
    )(et, w_t, b)


def kernel(x, emb_table, W, b):
    table_t = emb_table.T
    w_t = W.T
    et = _make_gather()(table_t, x.astype(jnp.int32))
    logits_t = _matmul(et, w_t, b)
    return (logits_t.T, et.T)
